# Initial kernel scaffold; baseline (speedup 1.0000x reference)
import jax
import jax.numpy as jnp
from jax.experimental import pallas as pl


def kernel(x, subject_id, W1, b1, A1, B1, W2, b2, A2, B2):
    raise NotImplementedError("write your pallas kernel here")


# speedup vs baseline: 2.4046x; 2.4046x over previous
import functools

import jax
import jax.numpy as jnp
from jax.experimental import pallas as pl
from jax.experimental.pallas import tpu as pltpu

RANK = 4
ALPHA = 1.0
NSUB = 16
DIN = 1024
DFF = 4096
TS = 512


def _fused(sid_ref, x_ref, W1_ref, b1_ref, A1_ref, B1_ref,
           W2_ref, b2_ref, A2_ref, B2_ref, out_ref):
    x = x_ref[0]
    scale = ALPHA / RANK
    base = jnp.dot(x, W1_ref[...], preferred_element_type=jnp.float32)
    lo = jnp.dot(x, A1_ref[0], preferred_element_type=jnp.float32)
    lo = jnp.dot(lo.astype(jnp.bfloat16), B1_ref[0],
                 preferred_element_type=jnp.float32)
    h = base + b1_ref[...] + scale * lo
    h = 0.5 * h * (1.0 + jax.lax.erf(h * 0.7071067811865476))
    hb = h.astype(jnp.bfloat16)
    base2 = jnp.dot(hb, W2_ref[...], preferred_element_type=jnp.float32)
    lo2 = jnp.dot(hb, A2_ref[0], preferred_element_type=jnp.float32)
    lo2 = jnp.dot(lo2.astype(jnp.bfloat16), B2_ref[0],
                  preferred_element_type=jnp.float32)
    out_ref[0] = base2 + b2_ref[...] + scale * lo2


def kernel(x, subject_id, W1, b1, A1, B1, W2, b2, A2, B2):
    B, S, _ = x.shape
    bf = jnp.bfloat16
    xb = x.astype(bf)
    W1b, W2b = W1.astype(bf), W2.astype(bf)
    A1b, B1b = A1.astype(bf), B1.astype(bf)
    A2b, B2b = A2.astype(bf), B2.astype(bf)
    b1r = b1.reshape(1, DFF)
    b2r = b2.reshape(1, DIN)
    sid = subject_id.astype(jnp.int32)

    grid = (B, S // TS)
    grid_spec = pltpu.PrefetchScalarGridSpec(
        num_scalar_prefetch=1,
        grid=grid,
        in_specs=[
            pl.BlockSpec((1, TS, DIN), lambda b, t, sid: (b, t, 0)),
            pl.BlockSpec((DIN, DFF), lambda b, t, sid: (0, 0)),
            pl.BlockSpec((1, DFF), lambda b, t, sid: (0, 0)),
            pl.BlockSpec((1, DIN, RANK), lambda b, t, sid: (sid[b], 0, 0)),
            pl.BlockSpec((1, RANK, DFF), lambda b, t, sid: (sid[b], 0, 0)),
            pl.BlockSpec((DFF, DIN), lambda b, t, sid: (0, 0)),
            pl.BlockSpec((1, DIN), lambda b, t, sid: (0, 0)),
            pl.BlockSpec((1, DFF, RANK), lambda b, t, sid: (sid[b], 0, 0)),
            pl.BlockSpec((1, RANK, DIN), lambda b, t, sid: (sid[b], 0, 0)),
        ],
        out_specs=pl.BlockSpec((1, TS, DIN), lambda b, t, sid: (b, t, 0)),
    )
    out = pl.pallas_call(
        _fused,
        grid_spec=grid_spec,
        out_shape=jax.ShapeDtypeStruct((B, S, DIN), jnp.float32),
        compiler_params=pltpu.CompilerParams(
            dimension_semantics=("arbitrary", "arbitrary"),
        ),
    )(sid, xb, W1b, b1r, A1b, B1b, W2b, b2r, A2b, B2b)
    return out
